# baseline (speedup 1.0000x reference)
import jax
import jax.numpy as jnp
from jax import lax
from jax.experimental import pallas as pl
from jax.experimental.pallas import tpu as pltpu
from jax.experimental.pallas import tpu_sc as plsc

VOCAB2 = 1000002
DIM = 64
B = 4096
L = 200
N = B * L

NUM_WORKERS = 32
B_PER_W = B // NUM_WORKERS
ROWS_PER_WORKER = B_PER_W * L
LC = 2
BLOCK_ROWS = LC * B_PER_W
NUM_BLOCKS = L // LC


def _gather_kernel(
    idx_hbm, table_hbm, out_hbm,
    slab, ridx0, ridx1, gbuf0, gbuf1, obuf0, obuf1,
    gsem0, gsem1, wsem0, wsem1,
):
    wid = lax.axis_index("s") * 2 + lax.axis_index("c")
    base = wid * ROWS_PER_WORKER
    ridx = (ridx0, ridx1)
    gbuf = (gbuf0, gbuf1)
    obuf = (obuf0, obuf1)
    gsem = (gsem0, gsem1)
    wsem = (wsem0, wsem1)

    lanes = lax.iota(jnp.int32, 16)

    pltpu.sync_copy(idx_hbm.at[pl.ds(base, ROWS_PER_WORKER)], slab)

    def build_ridx(blk, b):
        for lp in range(LC):
            l = blk * LC + lp

            def bbody(j, carry):
                src = (j * 16 + lanes) * L + l
                v = plsc.load_gather(slab, [src])
                ridx[b][pl.ds(lp * B_PER_W + j * 16, 16)] = v
                return carry

            lax.fori_loop(0, B_PER_W // 16, bbody, 0)

    def start_gather(b):
        pltpu.async_copy(table_hbm.at[ridx[b]], gbuf[b], gsem[b])

    def wait_gather(b):
        pltpu.make_async_copy(table_hbm.at[ridx[b]], gbuf[b], gsem[b]).wait()

    def transpose_block(b):
        for lp in range(LC):

            def tbody(d, carry):
                col = jnp.full((16,), 0, jnp.int32) + d
                for j in range(B_PER_W // 16):
                    row = lp * B_PER_W + j * 16 + lanes
                    v = plsc.load_gather(gbuf[b], [row, col])
                    obuf[b][lp, d, pl.ds(j * 16, 16)] = v
                return carry

            lax.fori_loop(0, DIM, tbody, 0)

    def start_store(blk, b):
        for lp in range(LC):
            pltpu.async_copy(
                obuf[b].at[lp],
                out_hbm.at[blk * LC + lp, :, pl.ds(wid * B_PER_W, B_PER_W)],
                wsem[b],
            )

    def wait_store(blk, b):
        for lp in range(LC):
            pltpu.make_async_copy(
                obuf[b].at[lp],
                out_hbm.at[blk * LC + lp, :, pl.ds(wid * B_PER_W, B_PER_W)],
                wsem[b],
            ).wait()

    build_ridx(0, 0)
    start_gather(0)

    def body(g, carry):
        for b in range(2):
            i = g * 2 + b

            @pl.when(i + 1 < NUM_BLOCKS)
            def _():
                build_ridx(i + 1, 1 - b)

            wait_gather(b)

            @pl.when(i + 1 < NUM_BLOCKS)
            def _():
                start_gather(1 - b)

            @pl.when(i >= 2)
            def _():
                wait_store(i - 2, b)

            transpose_block(b)
            start_store(i, b)
        return carry

    lax.fori_loop(0, NUM_BLOCKS // 2, body, 0)
    wait_store(NUM_BLOCKS - 2, 0)
    wait_store(NUM_BLOCKS - 1, 1)


@jax.jit
def kernel(word_input, weight_all):
    idx_flat = word_input.reshape(N)
    table128 = jnp.pad(weight_all, ((0, 0), (0, 128 - DIM)))
    mesh = plsc.VectorSubcoreMesh(core_axis_name="c", subcore_axis_name="s")
    out_t = pl.kernel(
        _gather_kernel,
        out_type=jax.ShapeDtypeStruct((L, DIM, B), jnp.float32),
        mesh=mesh,
        scratch_types=[
            pltpu.VMEM((ROWS_PER_WORKER,), jnp.int32),
            pltpu.VMEM((BLOCK_ROWS,), jnp.int32),
            pltpu.VMEM((BLOCK_ROWS,), jnp.int32),
            pltpu.VMEM((BLOCK_ROWS, 128), jnp.float32),
            pltpu.VMEM((BLOCK_ROWS, 128), jnp.float32),
            pltpu.VMEM((LC, DIM, B_PER_W), jnp.float32),
            pltpu.VMEM((LC, DIM, B_PER_W), jnp.float32),
            pltpu.SemaphoreType.DMA,
            pltpu.SemaphoreType.DMA,
            pltpu.SemaphoreType.DMA,
            pltpu.SemaphoreType.DMA,
        ],
        compiler_params=pltpu.CompilerParams(
            use_tc_tiling_on_sc=True, needs_layout_passes=False
        ),
    )(idx_flat, table128)
    return out_t.transpose(2, 0, 1)


# speedup vs baseline: 1.7528x; 1.7528x over previous
import jax
import jax.numpy as jnp
from jax import lax
from jax.experimental import pallas as pl
from jax.experimental.pallas import tpu as pltpu
from jax.experimental.pallas import tpu_sc as plsc

VOCAB2 = 1000002
DIM = 64
B = 4096
L = 200
N = B * L

NUM_WORKERS = 32
ROWS_PER_WORKER = N // NUM_WORKERS
CHUNK = 400
NUM_CHUNKS = ROWS_PER_WORKER // CHUNK
NBUF = 2


def _gather_kernel(idx_hbm, table_hbm, out_hbm, idx_v, rows0, rows1, sem0, sem1):
    wid = lax.axis_index("s") * 2 + lax.axis_index("c")
    base = wid * ROWS_PER_WORKER
    rows = (rows0, rows1)
    sems = (sem0, sem1)

    pltpu.sync_copy(idx_hbm.at[pl.ds(base, ROWS_PER_WORKER)], idx_v)

    def start_gather(i, b):
        pltpu.async_copy(
            table_hbm.at[idx_v.at[pl.ds(i * CHUNK, CHUNK)]], rows[b], sems[b]
        )

    def wait_gather(i, b):
        pltpu.make_async_copy(
            table_hbm.at[idx_v.at[pl.ds(i * CHUNK, CHUNK)]], rows[b], sems[b]
        ).wait()

    for b in range(NBUF):
        start_gather(b, b)

    def body(g, carry):
        for b in range(NBUF):
            i = g * NBUF + b
            wait_gather(i, b)
            pltpu.sync_copy(rows[b], out_hbm.at[pl.ds(base + i * CHUNK, CHUNK)])
            nxt = i + NBUF

            @pl.when(nxt < NUM_CHUNKS)
            def _():
                start_gather(nxt, b)

        return carry

    lax.fori_loop(0, NUM_CHUNKS // NBUF, body, 0)


@jax.jit
def kernel(word_input, weight_all):
    idx_flat = word_input.reshape(N)
    table128 = jnp.pad(weight_all, ((0, 0), (0, 128 - DIM)))
    mesh = plsc.VectorSubcoreMesh(core_axis_name="c", subcore_axis_name="s")
    out128 = pl.kernel(
        _gather_kernel,
        out_type=jax.ShapeDtypeStruct((N, 128), jnp.float32),
        mesh=mesh,
        scratch_types=[
            pltpu.VMEM((ROWS_PER_WORKER,), jnp.int32),
            pltpu.VMEM((CHUNK, 128), jnp.float32),
            pltpu.VMEM((CHUNK, 128), jnp.float32),
            pltpu.SemaphoreType.DMA,
            pltpu.SemaphoreType.DMA,
        ],
        compiler_params=pltpu.CompilerParams(use_tc_tiling_on_sc=True),
    )(idx_flat, table128)
    return out128[:, :DIM].reshape(B, L, DIM)
